# baseline (speedup 1.0000x reference)
import functools

import jax
import jax.numpy as jnp
from jax import lax
from jax.experimental import pallas as pl
from jax.experimental.pallas import tpu as pltpu
from jax.experimental.pallas import tpu_sc as plsc

_NUM_SEG = 256
_N = 100000
_D = 128
_BLK = 10000
_NBLK = _N // _BLK

_NW = 32
_CH = 128
_BIG = 13
_ROWS_BIG = 25 * _CH
_ROWS_SMALL = 24 * _CH
_EPI_BASE = 99968
_EPI_ROWS = 32
_ROW = _D + 16


def _score_body(x_ref, w1_ref, b1_ref, w2_ref, s_ref):
    x = x_ref[...]
    h = jnp.tanh(
        jnp.dot(x.astype(jnp.bfloat16), w1_ref[...],
                preferred_element_type=jnp.float32) + b1_ref[...])
    s_ref[...] = jnp.dot(h.astype(jnp.bfloat16), w2_ref[...],
                         preferred_element_type=jnp.float32)


def _scores(x, W1, b1, W2):
    return pl.pallas_call(
        _score_body,
        grid=(_NBLK,),
        in_specs=[
            pl.BlockSpec((_BLK, _D), lambda i: (i, 0)),
            pl.BlockSpec((_D, 64), lambda i: (0, 0)),
            pl.BlockSpec((1, 64), lambda i: (0, 0)),
            pl.BlockSpec((64, 1), lambda i: (0, 0)),
        ],
        out_specs=pl.BlockSpec((_BLK, 1), lambda i: (i, 0)),
        out_shape=jax.ShapeDtypeStruct((_N, 1), jnp.float32),
        compiler_params=pltpu.CompilerParams(
            dimension_semantics=("arbitrary",),
        ),
    )(x, W1.astype(jnp.bfloat16), b1.reshape(1, 64).astype(jnp.float32),
      W2.astype(jnp.bfloat16))


def _sc_pool_build():
    mesh = plsc.VectorSubcoreMesh(core_axis_name="c", subcore_axis_name="s")

    @functools.partial(
        pl.kernel,
        mesh=mesh,
        out_type=[
            jax.ShapeDtypeStruct((_NW, _NUM_SEG * _ROW), jnp.float32),
        ],
        scratch_types=[
            pltpu.VMEM((_NUM_SEG * _ROW,), jnp.float32),
            pltpu.VMEM((_CH * _D,), jnp.float32),
            pltpu.VMEM((_CH,), jnp.float32),
            pltpu.VMEM((_CH + 16,), jnp.float32),
            pltpu.VMEM((_CH + 16,), jnp.float32),
        ],
    )
    def pool(xf_hbm, s_hbm, b_hbm, part_hbm,
             acc, xc, sc, bc, exc):
        cid = lax.axis_index("c")
        sid = lax.axis_index("s")
        wid = cid * 16 + sid

        zero16 = jnp.zeros((16,), jnp.float32)
        lanes = lax.broadcasted_iota(jnp.int32, (16,), 0)
        unit16 = (lanes == 0).astype(jnp.float32)

        def _zero_acc(g, carry):
            acc[pl.ds(g * 16, 16)] = zero16
            return carry
        lax.fori_loop(0, (_NUM_SEG * _ROW) // 16, _zero_acc, 0)

        base = jnp.where(wid < _BIG, _ROWS_BIG * wid,
                         _BIG * _ROWS_BIG + _ROWS_SMALL * (wid - _BIG))
        nchunks = jnp.where(wid < _BIG, 25, 24)

        def _row_update(r, carry):
            seg = bc[pl.ds(r, 16)][0].astype(jnp.int32)
            wv = zero16 + exc[pl.ds(r, 16)][0]
            rb = r * _D
            sb = seg * _ROW
            for k in range(_D // 16):
                xv = xc[pl.ds(rb + k * 16, 16)]
                acc[pl.ds(sb + k * 16, 16)] += xv * wv
            acc[pl.ds(sb + _D, 16)] += wv
            return carry

        def _load_chunk(row0, nrows):
            pltpu.sync_copy(xf_hbm.at[pl.ds(row0 * _D, nrows * _D)],
                            xc.at[pl.ds(0, nrows * _D)])
            pltpu.sync_copy(s_hbm.at[pl.ds(row0, nrows)],
                            sc.at[pl.ds(0, nrows)])
            pltpu.sync_copy(b_hbm.at[pl.ds(row0, nrows)],
                            bc.at[pl.ds(0, nrows)])
            for g in range(nrows // 16):
                exc[pl.ds(g * 16, 16)] = jnp.exp(sc[pl.ds(g * 16, 16)])

        def _chunk(c, carry):
            _load_chunk(base + c * _CH, _CH)
            lax.fori_loop(0, _CH, _row_update, 0)
            return carry
        lax.fori_loop(0, nchunks, _chunk, 0)

        @pl.when(wid == _NW - 1)
        def _epilogue():
            _load_chunk(_EPI_BASE, _EPI_ROWS)
            lax.fori_loop(0, _EPI_ROWS, _row_update, 0)

        pltpu.sync_copy(acc, part_hbm.at[wid])

    return pool


_sc_pool = _sc_pool_build()


def _combine_body(part_ref, out_ref):
    a = jnp.sum(part_ref[...].reshape(_NW, _NUM_SEG, _ROW), axis=0)
    p = a[:, :_D]
    d = jnp.sum(a[:, _D:], axis=1, keepdims=True) * (1.0 / 16.0)
    out_ref[...] = p * (1.0 / (d + 1e-16))


def _combine(part):
    return pl.pallas_call(
        _combine_body,
        out_shape=jax.ShapeDtypeStruct((_NUM_SEG, _D), jnp.float32),
    )(part)


@jax.jit
def kernel(x, batch, W1, b1, W2, b2):
    del b2
    s = _scores(x, W1, b1, W2).reshape(_N)
    bt = batch.astype(jnp.int32).astype(jnp.float32)
    part, = _sc_pool(x.reshape(_N * _D), s, bt)
    return _combine(part)


# speedup vs baseline: 2.8166x; 2.8166x over previous
import jax
import jax.numpy as jnp
from jax.experimental import pallas as pl
from jax.experimental.pallas import tpu as pltpu

_NUM_SEG = 256
_N = 100000
_D = 128
_BLK = 5000
_NBLK = _N // _BLK


def _body(x_ref, bt_ref, w1_ref, b1_ref, w2_ref, b2_ref, out_ref,
          s_acc):
    i = pl.program_id(0)

    @pl.when(i == 0)
    def _init():
        s_acc[...] = jnp.zeros_like(s_acc)

    x = x_ref[...]
    xb = x.astype(jnp.bfloat16)
    h = jnp.tanh(
        jnp.dot(xb, w1_ref[...], preferred_element_type=jnp.float32)
        + b1_ref[...])
    s = (jnp.dot(h.astype(jnp.bfloat16), w2_ref[...],
                 preferred_element_type=jnp.float32)
         + b2_ref[...])
    ex = jnp.exp(s)

    bt = bt_ref[...]
    seg_ids = jax.lax.broadcasted_iota(jnp.int16, (_BLK, _NUM_SEG), 1)
    oh = jnp.where(seg_ids == bt,
                   jnp.bfloat16(1), jnp.bfloat16(0))

    ones = jnp.ones((_BLK, 1), jnp.float32)
    xa = jnp.concatenate([x, ones], axis=1)
    xe = (xa * ex).astype(jnp.bfloat16)
    s_acc[...] += jax.lax.dot_general(
        oh, xe, (((0,), (0,)), ((), ())),
        preferred_element_type=jnp.float32)

    @pl.when(i == _NBLK - 1)
    def _finish():
        inv = 1.0 / (s_acc[:, 128:129] + 1e-16)
        out_ref[...] = s_acc[:, :128] * inv


@jax.jit
def kernel(x, batch, W1, b1, W2, b2):
    bt2 = batch.astype(jnp.int16).reshape(_N, 1)
    b1r = b1.reshape(1, 64).astype(jnp.float32)
    b2r = b2.reshape(1, 1).astype(jnp.float32)
    w1b = W1.astype(jnp.bfloat16)
    w2b = W2.astype(jnp.bfloat16)
    out = pl.pallas_call(
        _body,
        grid=(_NBLK,),
        in_specs=[
            pl.BlockSpec((_BLK, _D), lambda i: (i, 0)),
            pl.BlockSpec((_BLK, 1), lambda i: (i, 0)),
            pl.BlockSpec((_D, 64), lambda i: (0, 0)),
            pl.BlockSpec((1, 64), lambda i: (0, 0)),
            pl.BlockSpec((64, 1), lambda i: (0, 0)),
            pl.BlockSpec((1, 1), lambda i: (0, 0)),
        ],
        out_specs=pl.BlockSpec((_NUM_SEG, _D), lambda i: (0, 0)),
        out_shape=jax.ShapeDtypeStruct((_NUM_SEG, _D), jnp.float32),
        scratch_shapes=[
            pltpu.VMEM((_NUM_SEG, _D + 1), jnp.float32),
        ],
        compiler_params=pltpu.CompilerParams(
            dimension_semantics=("arbitrary",),
        ),
    )(x, bt2, w1b, b1r, w2b, b2r)
    return out


# speedup vs baseline: 3.4094x; 1.2105x over previous
import jax
import jax.numpy as jnp
from jax.experimental import pallas as pl
from jax.experimental.pallas import tpu as pltpu

_NUM_SEG = 256
_N = 100000
_D = 128
_BLK = 5000
_NBLK = _N // _BLK


def _body(x_ref, bt_ref, w1_ref, b1_ref, w2_ref, b2_ref, out_ref,
          s_acc, d_acc):
    i = pl.program_id(0)

    @pl.when(i == 0)
    def _init():
        s_acc[...] = jnp.zeros_like(s_acc)
        d_acc[...] = jnp.zeros_like(d_acc)

    x = x_ref[...]
    h = jnp.tanh(
        jnp.dot(x, w1_ref[...], preferred_element_type=jnp.float32)
        + b1_ref[...])
    s = (jnp.dot(h, w2_ref[...], preferred_element_type=jnp.float32)
         + b2_ref[...])
    ex = jnp.exp(s)

    bt = bt_ref[...]
    seg_ids = jax.lax.broadcasted_iota(jnp.int32, (_BLK, _NUM_SEG), 1)
    oh = (seg_ids == bt).astype(jnp.float32)

    xe = x * ex
    s_acc[...] += jax.lax.dot_general(
        oh, xe, (((0,), (0,)), ((), ())),
        preferred_element_type=jnp.float32)
    d_acc[...] += jnp.sum(oh * ex, axis=0, keepdims=True)

    @pl.when(i == _NBLK - 1)
    def _finish():
        inv = 1.0 / (d_acc[...] + 1e-16)
        r = jax.lax.broadcasted_iota(jnp.int32, (_NUM_SEG, _NUM_SEG), 0)
        c = jax.lax.broadcasted_iota(jnp.int32, (_NUM_SEG, _NUM_SEG), 1)
        diag_inv = jnp.where(r == c, inv, 0.0)
        out_ref[...] = jnp.dot(diag_inv, s_acc[...],
                               preferred_element_type=jnp.float32)


@jax.jit
def kernel(x, batch, W1, b1, W2, b2):
    bt2 = batch.astype(jnp.int32).reshape(_N, 1)
    b1r = b1.reshape(1, 64).astype(jnp.float32)
    b2r = b2.reshape(1, 1).astype(jnp.float32)
    out = pl.pallas_call(
        _body,
        grid=(_NBLK,),
        in_specs=[
            pl.BlockSpec((_BLK, _D), lambda i: (i, 0)),
            pl.BlockSpec((_BLK, 1), lambda i: (i, 0)),
            pl.BlockSpec((_D, 64), lambda i: (0, 0)),
            pl.BlockSpec((1, 64), lambda i: (0, 0)),
            pl.BlockSpec((64, 1), lambda i: (0, 0)),
            pl.BlockSpec((1, 1), lambda i: (0, 0)),
        ],
        out_specs=pl.BlockSpec((_NUM_SEG, _D), lambda i: (0, 0)),
        out_shape=jax.ShapeDtypeStruct((_NUM_SEG, _D), jnp.float32),
        scratch_shapes=[
            pltpu.VMEM((_NUM_SEG, _D), jnp.float32),
            pltpu.VMEM((1, _NUM_SEG), jnp.float32),
        ],
        compiler_params=pltpu.CompilerParams(
            dimension_semantics=("arbitrary",),
        ),
    )(x, bt2, W1, b1r, W2, b2r)
    return out
